# baseline (speedup 1.0000x reference)
import functools

import jax
import jax.numpy as jnp
from jax import lax
from jax.experimental import pallas as pl
from jax.experimental.pallas import tpu as pltpu
from jax.experimental.pallas import tpu_sc as plsc

N = 10000
E = 320000
D = 128
H = 64
OUT = 6
G = 64

NC = 2
NS = 16
NW = NC * NS
CH = 128
KCH = E // CH
NCHUNK = KCH // NW
NEXTRA = KCH - NCHUNK * NW
NBUF = 6
LG = 5
LS = NBUF - LG
N_PAD = 10240
ROWS_PT = N_PAD // NS

ROW_BLK = 1000
NBLK = (N // 2) // ROW_BLK


def _sc_scatter_add(table, ei3, zeros):
    mesh = plsc.VectorSubcoreMesh(core_axis_name="c", subcore_axis_name="s")

    @functools.partial(
        pl.kernel,
        out_type=jax.ShapeDtypeStruct((NC, N_PAD, H), jnp.float32),
        mesh=mesh,
        scratch_types=[
            pltpu.VMEM((NCHUNK, CH), jnp.int32),
            pltpu.VMEM((NCHUNK, CH), jnp.int32),
            pltpu.VMEM((1, CH), jnp.int32),
            pltpu.VMEM((1, CH), jnp.int32),
            [pltpu.VMEM((CH, H), jnp.float32)] * NBUF,
            pltpu.VMEM_SHARED((N_PAD, H), jnp.float32),
            [pltpu.SemaphoreType.DMA] * NBUF,
            [pltpu.SemaphoreType.DMA] * NBUF,
        ],
        compiler_params=pltpu.CompilerParams(use_tc_tiling_on_sc=False),
    )
    def k(table_hbm, ei_hbm, zeros_hbm, out_hbm,
          src_v, dst_v, esrc_v, edst_v, rows, acc_sh, gsems, ssems):
        c = lax.axis_index("c")
        s = lax.axis_index("s")
        w = c * NS + s
        r0 = s * ROWS_PT
        pltpu.sync_copy(ei_hbm.at[0, pl.ds(w * NCHUNK, NCHUNK)], src_v)
        pltpu.sync_copy(ei_hbm.at[1, pl.ds(w * NCHUNK, NCHUNK)], dst_v)
        for j in range(LG):
            pltpu.async_copy(table_hbm.at[src_v.at[j]], rows[j], gsems[j])
        pltpu.sync_copy(zeros_hbm.at[pl.ds(r0, ROWS_PT)],
                        acc_sh.at[pl.ds(r0, ROWS_PT)])
        plsc.subcore_barrier()


        def body(g, carry):
            for u in range(NBUF):
                j = NBUF * g + u
                un = (u + LG) % NBUF
                pltpu.make_async_copy(table_hbm.at[src_v.at[j]],
                                      rows[u], gsems[u]).wait()
                pltpu.async_copy(rows[u], acc_sh.at[dst_v.at[j]],
                                 ssems[u], add=True)

                @pl.when(j >= LS)
                def _():
                    pltpu.make_async_copy(rows[un], acc_sh.at[dst_v.at[0]],
                                          ssems[un]).wait()

                @pl.when(j + LG < NCHUNK)
                def _():
                    pltpu.async_copy(table_hbm.at[src_v.at[j + LG]],
                                     rows[un], gsems[un])

            return carry

        lax.fori_loop(0, NCHUNK // NBUF, body, 0)
        for j in range(NCHUNK - LS, NCHUNK):
            pltpu.make_async_copy(rows[j % NBUF], acc_sh.at[dst_v.at[0]],
                                  ssems[j % NBUF]).wait()

        @pl.when(w < NEXTRA)
        def _():
            base = NW * NCHUNK + w
            pltpu.sync_copy(ei_hbm.at[0, pl.ds(base, 1)], esrc_v)
            pltpu.sync_copy(ei_hbm.at[1, pl.ds(base, 1)], edst_v)
            pltpu.async_copy(table_hbm.at[esrc_v.at[0]], rows[0],
                             gsems[0]).wait()
            pltpu.sync_copy(rows[0], acc_sh.at[edst_v.at[0]], add=True)

        plsc.subcore_barrier()
        pltpu.sync_copy(acc_sh.at[pl.ds(r0, ROWS_PT)],
                        out_hbm.at[c, pl.ds(r0, ROWS_PT)])

    return k(table, ei3, zeros)


def _mm_first(x, W):
    def body(xt_ref, xb_ref, w_ref, o_ref):
        ye = jnp.dot(xt_ref[...], w_ref[...],
                     preferred_element_type=jnp.float32)
        yo = jnp.dot(xb_ref[...], w_ref[...],
                     preferred_element_type=jnp.float32)
        o_ref[...] = jnp.concatenate([ye, yo], axis=1)

    return pl.pallas_call(
        body,
        grid=(NBLK,),
        in_specs=[
            pl.BlockSpec((ROW_BLK, D), lambda i: (i, 0)),
            pl.BlockSpec((ROW_BLK, D), lambda i: (i + NBLK, 0)),
            pl.BlockSpec((D, H), lambda i: (0, 0)),
        ],
        out_specs=pl.BlockSpec((ROW_BLK, 2 * H), lambda i: (i, 0)),
        out_shape=jax.ShapeDtypeStruct((N // 2, 2 * H), jnp.float32),
    )(x, x, W)


def _mlp_after_agg1(y1p, parts, b1a2, W1b2, b1b2):
    def body(y_ref, p_ref, ba_ref, w_ref, bb_ref, o_ref):
        t = y_ref[...] + p_ref[0] + p_ref[1] + ba_ref[...]
        t = jnp.maximum(t, 0.0)
        t = jnp.dot(t, w_ref[...], preferred_element_type=jnp.float32)
        o_ref[...] = jnp.maximum(t + bb_ref[...], 0.0)

    return pl.pallas_call(
        body,
        grid=(NBLK,),
        in_specs=[
            pl.BlockSpec((ROW_BLK, 2 * H), lambda i: (i, 0)),
            pl.BlockSpec((NC, ROW_BLK, 2 * H), lambda i: (0, i, 0)),
            pl.BlockSpec((1, 2 * H), lambda i: (0, 0)),
            pl.BlockSpec((2 * H, 2 * H), lambda i: (0, 0)),
            pl.BlockSpec((1, 2 * H), lambda i: (0, 0)),
        ],
        out_specs=pl.BlockSpec((ROW_BLK, 2 * H), lambda i: (i, 0)),
        out_shape=jax.ShapeDtypeStruct((N // 2, 2 * H), jnp.float32),
    )(y1p, parts, b1a2, W1b2, b1b2)


def _mlp2_pool_final(hp, parts, batchE, batchO, b2a2, W2a2, b2b2, W2b2,
                     Wl, bl2):
    def body(h_ref, q_ref, bE_ref, bO_ref, ba_ref, wa_ref, bb_ref, wb_ref,
             wl_ref, bl_ref, o_ref, acc, cnt):
        i = pl.program_id(0)

        @pl.when(i == 0)
        def _():
            acc[...] = jnp.zeros_like(acc)
            cnt[...] = jnp.zeros_like(cnt)

        t = h_ref[...] + q_ref[0] + q_ref[1]
        t = jnp.dot(t, wa_ref[...], preferred_element_type=jnp.float32)
        t = jnp.maximum(t + ba_ref[...], 0.0)
        o2 = jnp.dot(t, wb_ref[...], preferred_element_type=jnp.float32)
        o2 = o2 + bb_ref[...]
        gi = lax.broadcasted_iota(jnp.int32, (G, ROW_BLK), 0)
        ohE = (jnp.broadcast_to(bE_ref[0], (G, ROW_BLK)) == gi)
        ohO = (jnp.broadcast_to(bO_ref[0], (G, ROW_BLK)) == gi)
        ohE = ohE.astype(jnp.float32)
        ohO = ohO.astype(jnp.float32)
        acc[...] += (jnp.dot(ohE, o2[:, :H],
                             preferred_element_type=jnp.float32)
                     + jnp.dot(ohO, o2[:, H:],
                               preferred_element_type=jnp.float32))
        cnt[...] += (jnp.sum(ohE, axis=1, keepdims=True)
                     + jnp.sum(ohO, axis=1, keepdims=True))

        @pl.when(i == pl.num_programs(0) - 1)
        def _():
            pooled = acc[...] / jnp.maximum(cnt[...], 1.0)
            o_ref[...] = (jnp.dot(pooled, wl_ref[...],
                                  preferred_element_type=jnp.float32)
                          + bl_ref[...])

    return pl.pallas_call(
        body,
        grid=(NBLK,),
        in_specs=[
            pl.BlockSpec((ROW_BLK, 2 * H), lambda i: (i, 0)),
            pl.BlockSpec((NC, ROW_BLK, 2 * H), lambda i: (0, i, 0)),
            pl.BlockSpec((1, 1, ROW_BLK), lambda i: (i, 0, 0)),
            pl.BlockSpec((1, 1, ROW_BLK), lambda i: (i, 0, 0)),
            pl.BlockSpec((1, 2 * H), lambda i: (0, 0)),
            pl.BlockSpec((2 * H, 2 * H), lambda i: (0, 0)),
            pl.BlockSpec((1, 2 * H), lambda i: (0, 0)),
            pl.BlockSpec((2 * H, 2 * H), lambda i: (0, 0)),
            pl.BlockSpec((H, OUT), lambda i: (0, 0)),
            pl.BlockSpec((1, OUT), lambda i: (0, 0)),
        ],
        out_specs=pl.BlockSpec((G, OUT), lambda i: (0, 0)),
        out_shape=jax.ShapeDtypeStruct((G, OUT), jnp.float32),
        scratch_shapes=[
            pltpu.VMEM((G, H), jnp.float32),
            pltpu.VMEM((G, 1), jnp.float32),
        ],
    )(hp, parts, batchE, batchO, b2a2, W2a2, b2b2, W2b2, Wl, bl2)


def _blockdiag(W):
    Z = jnp.zeros_like(W)
    return jnp.concatenate(
        [jnp.concatenate([W, Z], axis=1),
         jnp.concatenate([Z, W], axis=1)], axis=0)


def _pairb(b):
    return jnp.concatenate([b, b]).reshape(1, 2 * H)


def kernel(x, ei, batch, W1a, b1a, W1b, b1b, W2a, b2a, W2b, b2b, Wl, bl):
    ei3 = (ei * 2 - jnp.where(ei >= N // 2, N - 1, 0)).reshape(2, KCH, CH)
    zeros = jnp.zeros((N_PAD, H), jnp.float32)
    batchE = batch[: N // 2].reshape(NBLK, 1, ROW_BLK)
    batchO = batch[N // 2:].reshape(NBLK, 1, ROW_BLK)
    W1b2 = _blockdiag(W1b)
    W2a2 = _blockdiag(W2a)
    W2b2 = _blockdiag(W2b)
    b1a2 = _pairb(b1a)
    b1b2 = _pairb(b1b)
    b2a2 = _pairb(b2a)
    b2b2 = _pairb(b2b)
    bl2 = bl.reshape(1, OUT)

    y1p = _mm_first(x, W1a)
    parts1 = _sc_scatter_add(y1p.reshape(N, H), ei3, zeros)
    parts1 = parts1.reshape(NC, N_PAD // 2, 2 * H)
    hp = _mlp_after_agg1(y1p, parts1, b1a2, W1b2, b1b2)
    parts2 = _sc_scatter_add(hp.reshape(N, H), ei3, zeros)
    parts2 = parts2.reshape(NC, N_PAD // 2, 2 * H)
    return _mlp2_pool_final(hp, parts2, batchE, batchO, b2a2, W2a2, b2b2,
                            W2b2, Wl, bl2)


# speedup vs baseline: 1.0103x; 1.0103x over previous
import functools

import jax
import jax.numpy as jnp
from jax import lax
from jax.experimental import pallas as pl
from jax.experimental.pallas import tpu as pltpu
from jax.experimental.pallas import tpu_sc as plsc

N = 10000
E = 320000
D = 128
H = 64
OUT = 6
G = 64

NC = 2
NS = 16
NW = NC * NS
CH = 128
KCH = E // CH
NCHUNK = KCH // NW
NEXTRA = KCH - NCHUNK * NW
NBUF = 6
LG = 4
LS = NBUF - LG
N_PAD = 10240
ROWS_PT = N_PAD // NS

ROW_BLK = 1000
NBLK = (N // 2) // ROW_BLK


def _sc_scatter_add(table, ei3, zeros):
    mesh = plsc.VectorSubcoreMesh(core_axis_name="c", subcore_axis_name="s")

    @functools.partial(
        pl.kernel,
        out_type=jax.ShapeDtypeStruct((NC, N_PAD, H), jnp.float32),
        mesh=mesh,
        scratch_types=[
            pltpu.VMEM((NCHUNK, CH), jnp.int32),
            pltpu.VMEM((NCHUNK, CH), jnp.int32),
            pltpu.VMEM((1, CH), jnp.int32),
            pltpu.VMEM((1, CH), jnp.int32),
            [pltpu.VMEM((CH, H), jnp.float32)] * NBUF,
            pltpu.VMEM_SHARED((N_PAD, H), jnp.float32),
            [pltpu.SemaphoreType.DMA] * NBUF,
            [pltpu.SemaphoreType.DMA] * NBUF,
        ],
        compiler_params=pltpu.CompilerParams(use_tc_tiling_on_sc=False),
    )
    def k(table_hbm, ei_hbm, zeros_hbm, out_hbm,
          src_v, dst_v, esrc_v, edst_v, rows, acc_sh, gsems, ssems):
        c = lax.axis_index("c")
        s = lax.axis_index("s")
        w = c * NS + s
        r0 = s * ROWS_PT
        pltpu.sync_copy(ei_hbm.at[pl.ds(w * NCHUNK, NCHUNK), 0], src_v)
        pltpu.sync_copy(ei_hbm.at[pl.ds(w * NCHUNK, NCHUNK), 1], dst_v)
        for j in range(LG):
            pltpu.async_copy(table_hbm.at[src_v.at[j]], rows[j], gsems[j])
        pltpu.sync_copy(zeros_hbm.at[pl.ds(r0, ROWS_PT)],
                        acc_sh.at[pl.ds(r0, ROWS_PT)])
        plsc.subcore_barrier()


        def body(g, carry):
            for u in range(NBUF):
                j = NBUF * g + u
                un = (u + LG) % NBUF
                pltpu.make_async_copy(table_hbm.at[src_v.at[j]],
                                      rows[u], gsems[u]).wait()
                pltpu.async_copy(rows[u], acc_sh.at[dst_v.at[j]],
                                 ssems[u], add=True)

                @pl.when(j >= LS)
                def _():
                    pltpu.make_async_copy(rows[un], acc_sh.at[dst_v.at[0]],
                                          ssems[un]).wait()

                @pl.when(j + LG < NCHUNK)
                def _():
                    pltpu.async_copy(table_hbm.at[src_v.at[j + LG]],
                                     rows[un], gsems[un])

            return carry

        lax.fori_loop(0, NCHUNK // NBUF, body, 0)
        for j in range(NCHUNK - LS, NCHUNK):
            pltpu.make_async_copy(rows[j % NBUF], acc_sh.at[dst_v.at[0]],
                                  ssems[j % NBUF]).wait()

        @pl.when(w < NEXTRA)
        def _():
            base = NW * NCHUNK + w
            pltpu.sync_copy(ei_hbm.at[pl.ds(base, 1), 0], esrc_v)
            pltpu.sync_copy(ei_hbm.at[pl.ds(base, 1), 1], edst_v)
            pltpu.async_copy(table_hbm.at[esrc_v.at[0]], rows[0],
                             gsems[0]).wait()
            pltpu.sync_copy(rows[0], acc_sh.at[edst_v.at[0]], add=True)

        plsc.subcore_barrier()
        pltpu.sync_copy(acc_sh.at[pl.ds(r0, ROWS_PT)],
                        out_hbm.at[c, pl.ds(r0, ROWS_PT)])

    return k(table, ei3, zeros)


def _mm_first(x, W):
    def body(xt_ref, xb_ref, w_ref, o_ref):
        ye = jnp.dot(xt_ref[...], w_ref[...],
                     preferred_element_type=jnp.float32)
        yo = jnp.dot(xb_ref[...], w_ref[...],
                     preferred_element_type=jnp.float32)
        o_ref[...] = jnp.concatenate([ye, yo], axis=1)

    return pl.pallas_call(
        body,
        grid=(NBLK,),
        in_specs=[
            pl.BlockSpec((ROW_BLK, D), lambda i: (i, 0)),
            pl.BlockSpec((ROW_BLK, D), lambda i: (i + NBLK, 0)),
            pl.BlockSpec((D, H), lambda i: (0, 0)),
        ],
        out_specs=pl.BlockSpec((ROW_BLK, 2 * H), lambda i: (i, 0)),
        out_shape=jax.ShapeDtypeStruct((N // 2, 2 * H), jnp.float32),
    )(x, x, W)


def _mlp_after_agg1(y1p, parts, b1a2, W1b2, b1b2):
    def body(y_ref, p_ref, ba_ref, w_ref, bb_ref, o_ref):
        t = y_ref[...] + p_ref[0] + p_ref[1] + ba_ref[...]
        t = jnp.maximum(t, 0.0)
        t = jnp.dot(t, w_ref[...], preferred_element_type=jnp.float32)
        o_ref[...] = jnp.maximum(t + bb_ref[...], 0.0)

    return pl.pallas_call(
        body,
        grid=(NBLK,),
        in_specs=[
            pl.BlockSpec((ROW_BLK, 2 * H), lambda i: (i, 0)),
            pl.BlockSpec((NC, ROW_BLK, 2 * H), lambda i: (0, i, 0)),
            pl.BlockSpec((1, 2 * H), lambda i: (0, 0)),
            pl.BlockSpec((2 * H, 2 * H), lambda i: (0, 0)),
            pl.BlockSpec((1, 2 * H), lambda i: (0, 0)),
        ],
        out_specs=pl.BlockSpec((ROW_BLK, 2 * H), lambda i: (i, 0)),
        out_shape=jax.ShapeDtypeStruct((N // 2, 2 * H), jnp.float32),
    )(y1p, parts, b1a2, W1b2, b1b2)


def _mlp2_pool_final(hp, parts, batchE, batchO, b2a2, W2a2, b2b2, W2b2,
                     Wl, bl2):
    def body(h_ref, q_ref, bE_ref, bO_ref, ba_ref, wa_ref, bb_ref, wb_ref,
             wl_ref, bl_ref, o_ref, acc, cnt):
        i = pl.program_id(0)

        @pl.when(i == 0)
        def _():
            acc[...] = jnp.zeros_like(acc)
            cnt[...] = jnp.zeros_like(cnt)

        t = h_ref[...] + q_ref[0] + q_ref[1]
        t = jnp.dot(t, wa_ref[...], preferred_element_type=jnp.float32)
        t = jnp.maximum(t + ba_ref[...], 0.0)
        o2 = jnp.dot(t, wb_ref[...], preferred_element_type=jnp.float32)
        o2 = o2 + bb_ref[...]
        gi = lax.broadcasted_iota(jnp.int32, (G, ROW_BLK), 0)
        ohE = (jnp.broadcast_to(bE_ref[0], (G, ROW_BLK)) == gi)
        ohO = (jnp.broadcast_to(bO_ref[0], (G, ROW_BLK)) == gi)
        ohE = ohE.astype(jnp.float32)
        ohO = ohO.astype(jnp.float32)
        acc[...] += (jnp.dot(ohE, o2[:, :H],
                             preferred_element_type=jnp.float32)
                     + jnp.dot(ohO, o2[:, H:],
                               preferred_element_type=jnp.float32))
        cnt[...] += (jnp.sum(ohE, axis=1, keepdims=True)
                     + jnp.sum(ohO, axis=1, keepdims=True))

        @pl.when(i == pl.num_programs(0) - 1)
        def _():
            pooled = acc[...] / jnp.maximum(cnt[...], 1.0)
            o_ref[...] = (jnp.dot(pooled, wl_ref[...],
                                  preferred_element_type=jnp.float32)
                          + bl_ref[...])

    return pl.pallas_call(
        body,
        grid=(NBLK,),
        in_specs=[
            pl.BlockSpec((ROW_BLK, 2 * H), lambda i: (i, 0)),
            pl.BlockSpec((NC, ROW_BLK, 2 * H), lambda i: (0, i, 0)),
            pl.BlockSpec((1, 1, ROW_BLK), lambda i: (i, 0, 0)),
            pl.BlockSpec((1, 1, ROW_BLK), lambda i: (i, 0, 0)),
            pl.BlockSpec((1, 2 * H), lambda i: (0, 0)),
            pl.BlockSpec((2 * H, 2 * H), lambda i: (0, 0)),
            pl.BlockSpec((1, 2 * H), lambda i: (0, 0)),
            pl.BlockSpec((2 * H, 2 * H), lambda i: (0, 0)),
            pl.BlockSpec((H, OUT), lambda i: (0, 0)),
            pl.BlockSpec((1, OUT), lambda i: (0, 0)),
        ],
        out_specs=pl.BlockSpec((G, OUT), lambda i: (0, 0)),
        out_shape=jax.ShapeDtypeStruct((G, OUT), jnp.float32),
        scratch_shapes=[
            pltpu.VMEM((G, H), jnp.float32),
            pltpu.VMEM((G, 1), jnp.float32),
        ],
    )(hp, parts, batchE, batchO, b2a2, W2a2, b2b2, W2b2, Wl, bl2)


def _blockdiag(W):
    Z = jnp.zeros_like(W)
    return jnp.concatenate(
        [jnp.concatenate([W, Z], axis=1),
         jnp.concatenate([Z, W], axis=1)], axis=0)


def _pairb(b):
    return jnp.concatenate([b, b]).reshape(1, 2 * H)


def kernel(x, ei, batch, W1a, b1a, W1b, b1b, W2a, b2a, W2b, b2b, Wl, bl):
    eim = ei * 2 - jnp.where(ei >= N // 2, N - 1, 0)
    ei3 = eim.reshape(2, KCH, CH).transpose(1, 0, 2)
    zeros = jnp.zeros((N_PAD, H), jnp.float32)
    batchE = batch[: N // 2].reshape(NBLK, 1, ROW_BLK)
    batchO = batch[N // 2:].reshape(NBLK, 1, ROW_BLK)
    W1b2 = _blockdiag(W1b)
    W2a2 = _blockdiag(W2a)
    W2b2 = _blockdiag(W2b)
    b1a2 = _pairb(b1a)
    b1b2 = _pairb(b1b)
    b2a2 = _pairb(b2a)
    b2b2 = _pairb(b2b)
    bl2 = bl.reshape(1, OUT)

    y1p = _mm_first(x, W1a)
    parts1 = _sc_scatter_add(y1p.reshape(N, H), ei3, zeros)
    parts1 = parts1.reshape(NC, N_PAD // 2, 2 * H)
    hp = _mlp_after_agg1(y1p, parts1, b1a2, W1b2, b1b2)
    parts2 = _sc_scatter_add(hp.reshape(N, H), ei3, zeros)
    parts2 = parts2.reshape(NC, N_PAD // 2, 2 * H)
    return _mlp2_pool_final(hp, parts2, batchE, batchO, b2a2, W2a2, b2b2,
                            W2b2, Wl, bl2)
